# baseline (speedup 1.0000x reference)
import functools
import math

import jax
import jax.numpy as jnp
from jax import lax
from jax.experimental import pallas as pl
from jax.experimental.pallas import tpu as pltpu
from jax.experimental.pallas import tpu_sc as plsc

RATIO = 0.5
NC = 2
NS = 16
B = 512


def _round_up(x, m):
    return (x + m - 1) // m * m


@functools.partial(jax.jit, static_argnames=("n_dst", "F"))
def _msg_pass(h, r, c, w, zeros, *, n_dst, F):
    E_pad = r.shape[0]
    Bw = 8192 // F
    E_half = E_pad // NC
    E_ps = E_half // NS
    nwin = E_ps // Bw
    mesh = plsc.VectorSubcoreMesh(core_axis_name="c", subcore_axis_name="s")

    def body(h_hbm, r_hbm, c_hbm, w_hbm, z_hbm, out_hbm,
             ri_v, ci_v, w_v, rows_v, acc_sh, sem):
        ci = lax.axis_index("c")
        si = lax.axis_index("s")

        @pl.when(si == 0)
        def _():
            pltpu.sync_copy(z_hbm, acc_sh)

        plsc.subcore_barrier()

        base = ci * E_half + si * E_ps

        def win(wi, carry):
            off = base + wi * Bw
            pltpu.sync_copy(r_hbm.at[pl.ds(off, Bw)], ri_v)
            pltpu.sync_copy(c_hbm.at[pl.ds(off, Bw)], ci_v)
            pltpu.sync_copy(w_hbm.at[pl.ds(off, Bw)], w_v)
            descs = []
            for s in range(Bw // 16):
                idx16 = ri_v[pl.ds(s * 16, 16)]
                descs.append(pltpu.async_copy(
                    h_hbm.at[idx16], rows_v.at[pl.ds(s * 16, 16)], sem))
            for d in descs:
                d.wait()

            def mul(g, carry2):
                w16 = w_v[pl.ds(g * 16, 16)]
                for j in range(16):
                    ws = w16[j]
                    e = g * 16 + j
                    for fb in range(F // 16):
                        val = rows_v[e, pl.ds(fb * 16, 16)]
                        rows_v[e, pl.ds(fb * 16, 16)] = val * ws
                return carry2

            lax.fori_loop(0, Bw // 16, mul, 0)
            pltpu.sync_copy(rows_v, acc_sh.at[ci_v], add=True)
            return carry

        lax.fori_loop(0, nwin, win, 0)
        plsc.subcore_barrier()

        @pl.when(si == 0)
        def _():
            pltpu.sync_copy(acc_sh, out_hbm.at[ci])

    fn = pl.kernel(
        body,
        out_type=jax.ShapeDtypeStruct((NC, n_dst, F), jnp.float32),
        mesh=mesh,
        compiler_params=pltpu.CompilerParams(use_tc_tiling_on_sc=False),
        scratch_types=[
            pltpu.VMEM((Bw,), jnp.int32),
            pltpu.VMEM((Bw,), jnp.int32),
            pltpu.VMEM((Bw,), jnp.float32),
            pltpu.VMEM((Bw, F), jnp.float32),
            pltpu.VMEM_SHARED((n_dst, F), jnp.float32),
            pltpu.SemaphoreType.DMA,
        ],
    )
    return fn(h, r, c, w, zeros)


@functools.partial(jax.jit, static_argnames=("n_dst",))
def _deg_pass(c, w, zeros, *, n_dst):
    E_pad = c.shape[0]
    E_half = E_pad // NC
    E_ps = E_half // NS
    nwin = E_ps // B
    mesh = plsc.VectorSubcoreMesh(core_axis_name="c", subcore_axis_name="s")

    def body(c_hbm, w_hbm, z_hbm, out_hbm, ci_v, w_v, rows_v, acc_sh):
        ci = lax.axis_index("c")
        si = lax.axis_index("s")

        @pl.when(si == 0)
        def _():
            pltpu.sync_copy(z_hbm, acc_sh)

        plsc.subcore_barrier()

        base = ci * E_half + si * E_ps

        def win(wi, carry):
            off = base + wi * B
            pltpu.sync_copy(c_hbm.at[pl.ds(off, B)], ci_v)
            pltpu.sync_copy(w_hbm.at[pl.ds(off, B)], w_v)

            def fill(g, carry2):
                w16 = w_v[pl.ds(g * 16, 16)]
                for j in range(16):
                    rows_v[g * 16 + j, :] = jnp.broadcast_to(w16[j], (16,))
                return carry2

            lax.fori_loop(0, B // 16, fill, 0)
            pltpu.sync_copy(rows_v, acc_sh.at[ci_v], add=True)
            return carry

        lax.fori_loop(0, nwin, win, 0)
        plsc.subcore_barrier()

        @pl.when(si == 0)
        def _():
            pltpu.sync_copy(acc_sh, out_hbm.at[ci])

    fn = pl.kernel(
        body,
        out_type=jax.ShapeDtypeStruct((NC, n_dst, 16), jnp.float32),
        mesh=mesh,
        compiler_params=pltpu.CompilerParams(use_tc_tiling_on_sc=False),
        scratch_types=[
            pltpu.VMEM((B,), jnp.int32),
            pltpu.VMEM((B,), jnp.float32),
            pltpu.VMEM((B, 16), jnp.float32),
            pltpu.VMEM_SHARED((n_dst, 16), jnp.float32),
        ],
    )
    return fn(c, w, zeros)


def _gcn_conv_xla(x, ei, ew, mask, W, b):
    n = x.shape[0]
    h = x @ W
    w = ew * mask
    loop = jnp.arange(n)
    r = jnp.concatenate([ei[0], loop])
    c = jnp.concatenate([ei[1], loop])
    w2 = jnp.concatenate([w, jnp.ones((n,), x.dtype)])
    deg = jnp.zeros((n,), x.dtype).at[c].add(w2)
    safe = jnp.where(deg > 0, deg, 1.0)
    dis = jnp.where(deg > 0, 1.0 / jnp.sqrt(safe), 0.0)
    norm = dis[r] * w2 * dis[c]
    out = jnp.zeros((n, h.shape[1]), x.dtype).at[c].add(norm[:, None] * h[r])
    return out + b


def _graph_conv_xla(x, ei, mask, Wrel, brel, Wroot):
    n = x.shape[0]
    msg = x[ei[0]] * mask[:, None]
    aggr = jnp.zeros((n, x.shape[1]), x.dtype).at[ei[1]].add(msg)
    return aggr @ Wrel + brel + x @ Wroot


def _sag_pool_xla(x, ei, ew, mask, Wrel, brel, Wroot, ratio):
    n = x.shape[0]
    score = _graph_conv_xla(x, ei, mask, Wrel, brel, Wroot).reshape(-1)
    k = int(math.ceil(ratio * n))
    _, perm = jax.lax.top_k(score, k)
    xk = x[perm] * jnp.tanh(score[perm])[:, None]
    mapping = jnp.full((n,), -1, jnp.int32).at[perm].set(jnp.arange(k, dtype=jnp.int32))
    nr = mapping[ei[0]]
    nc = mapping[ei[1]]
    valid = (nr >= 0) & (nc >= 0)
    nmask = mask * valid.astype(x.dtype)
    nr = jnp.where(valid, nr, 0)
    nc = jnp.where(valid, nc, 0)
    return xk, jnp.stack([nr, nc]), ew, nmask


def _pad_edges(r, c, w, n, E_pad):
    E = r.shape[0]
    pad = E_pad - E
    pad_idx = (jnp.arange(pad, dtype=jnp.int32) % n)
    rp = jnp.concatenate([r, pad_idx])
    cp = jnp.concatenate([c, pad_idx])
    wp = jnp.concatenate([w, jnp.zeros((pad,), jnp.float32)])
    return rp, cp, wp


def _gcn_conv_sc(x, rp, cp, normp, inv_deg, W, b, zeros, n):
    h = x @ W
    F = h.shape[1]
    parts = _msg_pass(h, rp, cp, normp, zeros[:, :F], n_dst=n, F=F)
    out = parts[0] + parts[1] + h * inv_deg[:, None] + b
    return jax.nn.relu(out)


def kernel(x, edge_index, edge_attr, W1, b1, W2, b2, p1_Wrel, p1_brel, p1_Wroot, W3, b3, p2_Wrel, p2_brel, p2_Wroot, W4, b4, W5, b5):
    x = x.reshape(-1, 3)
    ei = edge_index.reshape(2, -1)
    ew = edge_attr.reshape(-1)
    E = ei.shape[1]
    mask = jnp.ones((E,), x.dtype)

    x1 = jax.nn.relu(_gcn_conv_xla(x, ei, ew, mask, W1, b1))
    x2 = jax.nn.relu(_gcn_conv_xla(x1, ei, ew, mask, W2, b2))
    x3, eiB, ew, maskB = _sag_pool_xla(x2, ei, ew, mask, p1_Wrel, p1_brel, p1_Wroot, RATIO)
    x4 = jax.nn.relu(_gcn_conv_xla(x3, eiB, ew, maskB, W3, b3))
    x5, eiC, ew, maskC = _sag_pool_xla(x4, eiB, ew, maskB, p2_Wrel, p2_brel, p2_Wroot, RATIO)

    n2 = x5.shape[0]
    E_pad = _round_up(E, NC * NS * B)
    w2e = ew * maskC
    r2, c2, w2p = _pad_edges(eiC[0], eiC[1], w2e, n2, E_pad)

    z2 = jnp.zeros((n2, 64), jnp.float32)
    z2d = jnp.zeros((n2, 16), jnp.float32)

    dparts = _deg_pass(c2, w2p, z2d, n_dst=n2)
    deg = 1.0 + dparts[0, :, 0] + dparts[1, :, 0]
    dis = lax.rsqrt(deg)
    inv_deg = 1.0 / deg
    normp = dis[r2] * w2p * dis[c2]

    x6 = _gcn_conv_sc(x5, r2, c2, normp, inv_deg, W4, b4, z2, n2)
    x7 = _gcn_conv_sc(x6, r2, c2, normp, inv_deg, W5, b5, z2, n2)
    return x7


# speedup vs baseline: 1.2121x; 1.2121x over previous
import functools
import math

import jax
import jax.numpy as jnp
from jax import lax
from jax.experimental import pallas as pl
from jax.experimental.pallas import tpu as pltpu
from jax.experimental.pallas import tpu_sc as plsc

RATIO = 0.5
NC = 2
NS = 16
B = 512


def _round_up(x, m):
    return (x + m - 1) // m * m


@functools.partial(jax.jit, static_argnames=("n_dst", "F"))
def _msg_pass(h, r, c, w, zeros, *, n_dst, F):
    E_pad = r.shape[0]
    Bw = 8192 // F
    E_half = E_pad // NC
    E_ps = E_half // NS
    nwin = E_ps // Bw
    mesh = plsc.VectorSubcoreMesh(core_axis_name="c", subcore_axis_name="s")

    def body(h_hbm, r_hbm, c_hbm, w_hbm, z_hbm, out_hbm,
             ri_v, ci_v, w_v, rows_v, acc_sh, sem):
        ci = lax.axis_index("c")
        si = lax.axis_index("s")

        @pl.when(si == 0)
        def _():
            pltpu.sync_copy(z_hbm, acc_sh)

        plsc.subcore_barrier()

        base = ci * E_half + si * E_ps

        def win(wi, carry):
            off = base + wi * Bw
            pltpu.sync_copy(r_hbm.at[pl.ds(off, Bw)], ri_v)
            pltpu.sync_copy(c_hbm.at[pl.ds(off, Bw)], ci_v)
            pltpu.sync_copy(w_hbm.at[pl.ds(off, Bw)], w_v)
            descs = []
            for s in range(Bw // 16):
                idx16 = ri_v[pl.ds(s * 16, 16)]
                descs.append(pltpu.async_copy(
                    h_hbm.at[idx16], rows_v.at[pl.ds(s * 16, 16)], sem))
            for d in descs:
                d.wait()

            def mul(g, carry2):
                w16 = w_v[pl.ds(g * 16, 16)]
                for j in range(16):
                    ws = w16[j]
                    e = g * 16 + j
                    for fb in range(F // 16):
                        val = rows_v[e, pl.ds(fb * 16, 16)]
                        rows_v[e, pl.ds(fb * 16, 16)] = val * ws
                return carry2

            lax.fori_loop(0, Bw // 16, mul, 0)
            pltpu.sync_copy(rows_v, acc_sh.at[ci_v], add=True)
            return carry

        lax.fori_loop(0, nwin, win, 0)
        plsc.subcore_barrier()

        @pl.when(si == 0)
        def _():
            pltpu.sync_copy(acc_sh, out_hbm.at[ci])

    fn = pl.kernel(
        body,
        out_type=jax.ShapeDtypeStruct((NC, n_dst, F), jnp.float32),
        mesh=mesh,
        compiler_params=pltpu.CompilerParams(use_tc_tiling_on_sc=False),
        scratch_types=[
            pltpu.VMEM((Bw,), jnp.int32),
            pltpu.VMEM((Bw,), jnp.int32),
            pltpu.VMEM((Bw,), jnp.float32),
            pltpu.VMEM((Bw, F), jnp.float32),
            pltpu.VMEM_SHARED((n_dst, F), jnp.float32),
            pltpu.SemaphoreType.DMA,
        ],
    )
    return fn(h, r, c, w, zeros)


@functools.partial(jax.jit, static_argnames=("n_dst",))
def _deg_pass(c, w, zeros, *, n_dst):
    E_pad = c.shape[0]
    E_half = E_pad // NC
    E_ps = E_half // NS
    nwin = E_ps // B
    mesh = plsc.VectorSubcoreMesh(core_axis_name="c", subcore_axis_name="s")

    def body(c_hbm, w_hbm, z_hbm, out_hbm, ci_v, w_v, rows_v, acc_sh):
        ci = lax.axis_index("c")
        si = lax.axis_index("s")

        @pl.when(si == 0)
        def _():
            pltpu.sync_copy(z_hbm, acc_sh)

        plsc.subcore_barrier()

        base = ci * E_half + si * E_ps

        def win(wi, carry):
            off = base + wi * B
            pltpu.sync_copy(c_hbm.at[pl.ds(off, B)], ci_v)
            pltpu.sync_copy(w_hbm.at[pl.ds(off, B)], w_v)

            def fill(g, carry2):
                w16 = w_v[pl.ds(g * 16, 16)]
                for j in range(16):
                    rows_v[g * 16 + j, :] = jnp.broadcast_to(w16[j], (16,))
                return carry2

            lax.fori_loop(0, B // 16, fill, 0)
            pltpu.sync_copy(rows_v, acc_sh.at[ci_v], add=True)
            return carry

        lax.fori_loop(0, nwin, win, 0)
        plsc.subcore_barrier()

        @pl.when(si == 0)
        def _():
            pltpu.sync_copy(acc_sh, out_hbm.at[ci])

    fn = pl.kernel(
        body,
        out_type=jax.ShapeDtypeStruct((NC, n_dst, 16), jnp.float32),
        mesh=mesh,
        compiler_params=pltpu.CompilerParams(use_tc_tiling_on_sc=False),
        scratch_types=[
            pltpu.VMEM((B,), jnp.int32),
            pltpu.VMEM((B,), jnp.float32),
            pltpu.VMEM((B, 16), jnp.float32),
            pltpu.VMEM_SHARED((n_dst, 16), jnp.float32),
        ],
    )
    return fn(c, w, zeros)


def _gcn_conv_xla(x, ei, ew, mask, W, b):
    n = x.shape[0]
    h = x @ W
    w = ew * mask
    loop = jnp.arange(n)
    r = jnp.concatenate([ei[0], loop])
    c = jnp.concatenate([ei[1], loop])
    w2 = jnp.concatenate([w, jnp.ones((n,), x.dtype)])
    deg = jnp.zeros((n,), x.dtype).at[c].add(w2)
    safe = jnp.where(deg > 0, deg, 1.0)
    dis = jnp.where(deg > 0, 1.0 / jnp.sqrt(safe), 0.0)
    norm = dis[r] * w2 * dis[c]
    out = jnp.zeros((n, h.shape[1]), x.dtype).at[c].add(norm[:, None] * h[r])
    return out + b


def _graph_conv_xla(x, ei, mask, Wrel, brel, Wroot):
    n = x.shape[0]
    msg = x[ei[0]] * mask[:, None]
    aggr = jnp.zeros((n, x.shape[1]), x.dtype).at[ei[1]].add(msg)
    return aggr @ Wrel + brel + x @ Wroot


def _sag_pool_xla(x, ei, ew, mask, Wrel, brel, Wroot, ratio):
    n = x.shape[0]
    score = _graph_conv_xla(x, ei, mask, Wrel, brel, Wroot).reshape(-1)
    k = int(math.ceil(ratio * n))
    _, perm = jax.lax.top_k(score, k)
    xk = x[perm] * jnp.tanh(score[perm])[:, None]
    mapping = jnp.full((n,), -1, jnp.int32).at[perm].set(jnp.arange(k, dtype=jnp.int32))
    nr = mapping[ei[0]]
    nc = mapping[ei[1]]
    valid = (nr >= 0) & (nc >= 0)
    nmask = mask * valid.astype(x.dtype)
    nr = jnp.where(valid, nr, 0)
    nc = jnp.where(valid, nc, 0)
    return xk, jnp.stack([nr, nc]), ew, nmask


def _pad_edges(r, c, w, n, E_pad):
    E = r.shape[0]
    pad = E_pad - E
    pad_idx = (jnp.arange(pad, dtype=jnp.int32) % n)
    rp = jnp.concatenate([r, pad_idx])
    cp = jnp.concatenate([c, pad_idx])
    wp = jnp.concatenate([w, jnp.zeros((pad,), jnp.float32)])
    return rp, cp, wp


def _gcn_conv_sc(x, rp, cp, wp, dis, inv_deg, W, b, zeros, n):
    h = x @ W
    F = h.shape[1]
    ht = dis[:, None] * h
    parts = _msg_pass(ht, rp, cp, wp, zeros[:, :F], n_dst=n, F=F)
    out = dis[:, None] * (parts[0] + parts[1]) + h * inv_deg[:, None] + b
    return jax.nn.relu(out)


def kernel(x, edge_index, edge_attr, W1, b1, W2, b2, p1_Wrel, p1_brel, p1_Wroot, W3, b3, p2_Wrel, p2_brel, p2_Wroot, W4, b4, W5, b5):
    x = x.reshape(-1, 3)
    ei = edge_index.reshape(2, -1)
    ew = edge_attr.reshape(-1)
    E = ei.shape[1]
    mask = jnp.ones((E,), x.dtype)

    x1 = jax.nn.relu(_gcn_conv_xla(x, ei, ew, mask, W1, b1))
    x2 = jax.nn.relu(_gcn_conv_xla(x1, ei, ew, mask, W2, b2))
    x3, eiB, ew, maskB = _sag_pool_xla(x2, ei, ew, mask, p1_Wrel, p1_brel, p1_Wroot, RATIO)
    x4 = jax.nn.relu(_gcn_conv_xla(x3, eiB, ew, maskB, W3, b3))
    x5, eiC, ew, maskC = _sag_pool_xla(x4, eiB, ew, maskB, p2_Wrel, p2_brel, p2_Wroot, RATIO)

    n2 = x5.shape[0]
    E_pad = _round_up(E, NC * NS * B)
    w2e = ew * maskC
    r2, c2, w2p = _pad_edges(eiC[0], eiC[1], w2e, n2, E_pad)

    z2 = jnp.zeros((n2, 64), jnp.float32)
    z2d = jnp.zeros((n2, 16), jnp.float32)

    dparts = _deg_pass(c2, w2p, z2d, n_dst=n2)
    deg = 1.0 + dparts[0, :, 0] + dparts[1, :, 0]
    dis = lax.rsqrt(deg)
    inv_deg = 1.0 / deg

    x6 = _gcn_conv_sc(x5, r2, c2, w2p, dis, inv_deg, W4, b4, z2, n2)
    x7 = _gcn_conv_sc(x6, r2, c2, w2p, dis, inv_deg, W5, b5, z2, n2)
    return x7


# speedup vs baseline: 1.4196x; 1.1712x over previous
import functools
import math

import jax
import jax.numpy as jnp
from jax import lax
from jax.experimental import pallas as pl
from jax.experimental.pallas import tpu as pltpu
from jax.experimental.pallas import tpu_sc as plsc

RATIO = 0.5
NC = 2
NS = 16
B = 512


def _round_up(x, m):
    return (x + m - 1) // m * m


@functools.partial(jax.jit, static_argnames=("n_dst", "F", "stage_src"))
def _msg_pass(h, r, c, w, zeros, *, n_dst, F, stage_src=False):
    E_pad = r.shape[0]
    Bw = 8192 // F
    E_half = E_pad // NC
    E_ps = E_half // NS
    nwin = E_ps // Bw
    mesh = plsc.VectorSubcoreMesh(core_axis_name="c", subcore_axis_name="s")

    def body(h_hbm, r_hbm, c_hbm, w_hbm, z_hbm, out_hbm,
             ri_v, ci_v, w_v, rows_v, acc_sh, *rest):
        if stage_src:
            h_sh, sem = rest
        else:
            (sem,) = rest
        ci = lax.axis_index("c")
        si = lax.axis_index("s")

        @pl.when(si == 0)
        def _():
            pltpu.sync_copy(z_hbm, acc_sh)
            if stage_src:
                pltpu.sync_copy(h_hbm, h_sh)

        plsc.subcore_barrier()

        base = ci * E_half + si * E_ps

        def win(wi, carry):
            off = base + wi * Bw
            pltpu.sync_copy(r_hbm.at[pl.ds(off, Bw)], ri_v)
            pltpu.sync_copy(c_hbm.at[pl.ds(off, Bw)], ci_v)
            pltpu.sync_copy(w_hbm.at[pl.ds(off, Bw)], w_v)
            descs = []
            src_ref = h_sh if stage_src else h_hbm
            for s in range(Bw // 16):
                idx16 = ri_v[pl.ds(s * 16, 16)]
                descs.append(pltpu.async_copy(
                    src_ref.at[idx16], rows_v.at[pl.ds(s * 16, 16)], sem))
            for d in descs:
                d.wait()

            def mul(g, carry2):
                w16 = w_v[pl.ds(g * 16, 16)]
                for j in range(16):
                    ws = w16[j]
                    e = g * 16 + j
                    for fb in range(F // 16):
                        val = rows_v[e, pl.ds(fb * 16, 16)]
                        rows_v[e, pl.ds(fb * 16, 16)] = val * ws
                return carry2

            lax.fori_loop(0, Bw // 16, mul, 0)
            pltpu.sync_copy(rows_v, acc_sh.at[ci_v], add=True)
            return carry

        lax.fori_loop(0, nwin, win, 0)
        plsc.subcore_barrier()

        @pl.when(si == 0)
        def _():
            pltpu.sync_copy(acc_sh, out_hbm.at[ci])

    fn = pl.kernel(
        body,
        out_type=jax.ShapeDtypeStruct((NC, n_dst, F), jnp.float32),
        mesh=mesh,
        compiler_params=pltpu.CompilerParams(use_tc_tiling_on_sc=False),
        scratch_types=[
            pltpu.VMEM((Bw,), jnp.int32),
            pltpu.VMEM((Bw,), jnp.int32),
            pltpu.VMEM((Bw,), jnp.float32),
            pltpu.VMEM((Bw, F), jnp.float32),
            pltpu.VMEM_SHARED((n_dst, F), jnp.float32),
        ] + ([pltpu.VMEM_SHARED(h.shape, jnp.float32)] if stage_src else [])
          + [pltpu.SemaphoreType.DMA],
    )
    return fn(h, r, c, w, zeros)


@functools.partial(jax.jit, static_argnames=("n_dst",))
def _deg_pass(c, w, zeros, *, n_dst):
    E_pad = c.shape[0]
    E_half = E_pad // NC
    E_ps = E_half // NS
    nwin = E_ps // B
    mesh = plsc.VectorSubcoreMesh(core_axis_name="c", subcore_axis_name="s")

    def body(c_hbm, w_hbm, z_hbm, out_hbm, ci_v, w_v, rows_v, acc_sh):
        ci = lax.axis_index("c")
        si = lax.axis_index("s")

        @pl.when(si == 0)
        def _():
            pltpu.sync_copy(z_hbm, acc_sh)

        plsc.subcore_barrier()

        base = ci * E_half + si * E_ps

        def win(wi, carry):
            off = base + wi * B
            pltpu.sync_copy(c_hbm.at[pl.ds(off, B)], ci_v)
            pltpu.sync_copy(w_hbm.at[pl.ds(off, B)], w_v)

            def fill(g, carry2):
                w16 = w_v[pl.ds(g * 16, 16)]
                for j in range(16):
                    rows_v[g * 16 + j, :] = jnp.broadcast_to(w16[j], (16,))
                return carry2

            lax.fori_loop(0, B // 16, fill, 0)
            pltpu.sync_copy(rows_v, acc_sh.at[ci_v], add=True)
            return carry

        lax.fori_loop(0, nwin, win, 0)
        plsc.subcore_barrier()

        @pl.when(si == 0)
        def _():
            pltpu.sync_copy(acc_sh, out_hbm.at[ci])

    fn = pl.kernel(
        body,
        out_type=jax.ShapeDtypeStruct((NC, n_dst, 16), jnp.float32),
        mesh=mesh,
        compiler_params=pltpu.CompilerParams(use_tc_tiling_on_sc=False),
        scratch_types=[
            pltpu.VMEM((B,), jnp.int32),
            pltpu.VMEM((B,), jnp.float32),
            pltpu.VMEM((B, 16), jnp.float32),
            pltpu.VMEM_SHARED((n_dst, 16), jnp.float32),
        ],
    )
    return fn(c, w, zeros)


def _gcn_conv_xla(x, ei, ew, mask, W, b):
    n = x.shape[0]
    h = x @ W
    w = ew * mask
    loop = jnp.arange(n)
    r = jnp.concatenate([ei[0], loop])
    c = jnp.concatenate([ei[1], loop])
    w2 = jnp.concatenate([w, jnp.ones((n,), x.dtype)])
    deg = jnp.zeros((n,), x.dtype).at[c].add(w2)
    safe = jnp.where(deg > 0, deg, 1.0)
    dis = jnp.where(deg > 0, 1.0 / jnp.sqrt(safe), 0.0)
    norm = dis[r] * w2 * dis[c]
    out = jnp.zeros((n, h.shape[1]), x.dtype).at[c].add(norm[:, None] * h[r])
    return out + b


def _graph_conv_xla(x, ei, mask, Wrel, brel, Wroot):
    n = x.shape[0]
    msg = x[ei[0]] * mask[:, None]
    aggr = jnp.zeros((n, x.shape[1]), x.dtype).at[ei[1]].add(msg)
    return aggr @ Wrel + brel + x @ Wroot


def _sag_pool_xla(x, ei, ew, mask, Wrel, brel, Wroot, ratio):
    n = x.shape[0]
    score = _graph_conv_xla(x, ei, mask, Wrel, brel, Wroot).reshape(-1)
    k = int(math.ceil(ratio * n))
    _, perm = jax.lax.top_k(score, k)
    xk = x[perm] * jnp.tanh(score[perm])[:, None]
    mapping = jnp.full((n,), -1, jnp.int32).at[perm].set(jnp.arange(k, dtype=jnp.int32))
    nr = mapping[ei[0]]
    nc = mapping[ei[1]]
    valid = (nr >= 0) & (nc >= 0)
    nmask = mask * valid.astype(x.dtype)
    nr = jnp.where(valid, nr, 0)
    nc = jnp.where(valid, nc, 0)
    return xk, jnp.stack([nr, nc]), ew, nmask


def _pad_edges(r, c, w, n, E_pad):
    E = r.shape[0]
    pad = E_pad - E
    pad_idx = (jnp.arange(pad, dtype=jnp.int32) % n)
    rp = jnp.concatenate([r, pad_idx])
    cp = jnp.concatenate([c, pad_idx])
    wp = jnp.concatenate([w, jnp.zeros((pad,), jnp.float32)])
    return rp, cp, wp


def _gcn_conv_sc(x, rp, cp, wp, dis, inv_deg, W, b, zeros, n):
    h = x @ W
    F = h.shape[1]
    ht = dis[:, None] * h
    parts = _msg_pass(ht, rp, cp, wp, zeros[:, :F], n_dst=n, F=F, stage_src=True)
    out = dis[:, None] * (parts[0] + parts[1]) + h * inv_deg[:, None] + b
    return jax.nn.relu(out)


def kernel(x, edge_index, edge_attr, W1, b1, W2, b2, p1_Wrel, p1_brel, p1_Wroot, W3, b3, p2_Wrel, p2_brel, p2_Wroot, W4, b4, W5, b5):
    x = x.reshape(-1, 3)
    ei = edge_index.reshape(2, -1)
    ew = edge_attr.reshape(-1)
    E = ei.shape[1]
    mask = jnp.ones((E,), x.dtype)

    x1 = jax.nn.relu(_gcn_conv_xla(x, ei, ew, mask, W1, b1))
    x2 = jax.nn.relu(_gcn_conv_xla(x1, ei, ew, mask, W2, b2))
    x3, eiB, ew, maskB = _sag_pool_xla(x2, ei, ew, mask, p1_Wrel, p1_brel, p1_Wroot, RATIO)
    x4 = jax.nn.relu(_gcn_conv_xla(x3, eiB, ew, maskB, W3, b3))
    x5, eiC, ew, maskC = _sag_pool_xla(x4, eiB, ew, maskB, p2_Wrel, p2_brel, p2_Wroot, RATIO)

    n2 = x5.shape[0]
    E_pad = _round_up(E, NC * NS * B)
    w2e = ew * maskC
    r2, c2, w2p = _pad_edges(eiC[0], eiC[1], w2e, n2, E_pad)

    z2 = jnp.zeros((n2, 64), jnp.float32)
    z2d = jnp.zeros((n2, 16), jnp.float32)

    dparts = _deg_pass(c2, w2p, z2d, n_dst=n2)
    deg = 1.0 + dparts[0, :, 0] + dparts[1, :, 0]
    dis = lax.rsqrt(deg)
    inv_deg = 1.0 / deg

    x6 = _gcn_conv_sc(x5, r2, c2, w2p, dis, inv_deg, W4, b4, z2, n2)
    x7 = _gcn_conv_sc(x6, r2, c2, w2p, dis, inv_deg, W5, b5, z2, n2)
    return x7
